# Initial kernel scaffold; baseline (speedup 1.0000x reference)
import jax
import jax.numpy as jnp
from jax.experimental import pallas as pl


def kernel(h, wg, bg, wd, bd):
    raise NotImplementedError("write your pallas kernel here")


# speedup vs baseline: 28.3164x; 28.3164x over previous
import functools

import jax
import jax.numpy as jnp
from jax import lax
from jax.experimental import pallas as pl
from jax.experimental.pallas import tpu as pltpu

D = 512
FAST = 16
SLOW = 64
B = 32
T = 128
STEPS = T - 3
_HI = lax.Precision.HIGHEST


def _scan_body(hT_ref, wgd_ref, bgd_ref, fidx_ref, sidx_ref, fused_ref,
               sused_ref, sc_ref):
    bgd = bgd_ref[...]
    wgd_bf = wgd_ref[...].astype(jnp.bfloat16)
    for k in range(8):
        blk = hT_ref[pl.ds(k * 16, 16)]
        v = blk.reshape(16 * B, D).astype(jnp.bfloat16)
        s = jnp.dot(v, wgd_bf,
                    preferred_element_type=jnp.float32) + bgd
        ws = jax.nn.sigmoid(s[:, 0:1])
        dd = s[:, 1:2]
        sc_ref[pl.ds(k * 16 * B, 16 * B), :] = jnp.concatenate([ws, dd], 1)

    iota16 = lax.broadcasted_iota(jnp.int32, (B, FAST), 1)
    iota64 = lax.broadcasted_iota(jnp.int32, (B, SLOW), 1)

    def step(t, carry):
        fast_score, fast_tok, slow_tok, nfast, ndem = carry
        s = sc_ref[pl.ds(t * B, B), :]
        w = s[:, 0:1] >= 0.4
        d = s[:, 1:2]
        full = nfast >= FAST
        m = jnp.min(fast_score, axis=1, keepdims=True)
        jmin = jnp.min(jnp.where(fast_score == m, iota16, FAST), axis=1,
                       keepdims=True)
        slot = jnp.where(full, jmin, nfast)
        onehot_f = (iota16 == slot) & w
        victim = jnp.sum(jnp.where(iota16 == jmin, fast_tok, 0), axis=1,
                         keepdims=True)
        demote = w & full
        ring = jnp.bitwise_and(ndem, SLOW - 1)
        onehot_s = (iota64 == ring) & demote
        slow_tok = jnp.where(onehot_s, victim, slow_tok)
        fast_score = jnp.where(onehot_f, d, fast_score)
        fast_tok = jnp.where(onehot_f, t, fast_tok)
        nfast = nfast + (w & ~full).astype(jnp.int32)
        ndem = ndem + demote.astype(jnp.int32)
        return fast_score, fast_tok, slow_tok, nfast, ndem

    init = (jnp.zeros((B, FAST), jnp.float32), jnp.zeros((B, FAST), jnp.int32),
            jnp.zeros((B, SLOW), jnp.int32), jnp.zeros((B, 1), jnp.int32),
            jnp.zeros((B, 1), jnp.int32))
    _, fast_tok, slow_tok, nfast, ndem = lax.fori_loop(0, STEPS, step, init)

    fused = iota16 < nfast
    sused = iota64 < jnp.minimum(ndem, SLOW)
    brow16 = lax.broadcasted_iota(jnp.int32, (B, FAST), 0)
    brow64 = lax.broadcasted_iota(jnp.int32, (B, SLOW), 0)
    fidx_ref[...] = jnp.where(fused, fast_tok * B + brow16, 0)
    sidx_ref[...] = jnp.where(sused, slow_tok * B + brow64, 0)
    fused_ref[...] = fused.astype(jnp.float32)
    sused_ref[...] = sused.astype(jnp.float32)


def _gather_body(h_ref, fidx_ref, sidx_ref, fused_ref, sused_ref,
                 fout_ref, sout_ref):
    b = pl.program_id(0)
    hb = h_ref[0]
    iota_tf = lax.broadcasted_iota(jnp.int32, (T, FAST), 0)
    iota_ts = lax.broadcasted_iota(jnp.int32, (T, SLOW), 0)
    ftok = fidx_ref[pl.ds(b, 1), :] // B
    stok = sidx_ref[pl.ds(b, 1), :] // B
    fmask = fused_ref[pl.ds(b, 1), :]
    smask = sused_ref[pl.ds(b, 1), :]
    pf = jnp.where(iota_tf == ftok, 1.0, 0.0) * fmask
    ps = jnp.where(iota_ts == stok, 1.0, 0.0) * smask
    dn = (((0,), (0,)), ((), ()))
    fout_ref[0] = lax.dot_general(pf, hb, dn, precision=_HI)
    sout_ref[0] = lax.dot_general(ps, hb, dn, precision=_HI)


@jax.jit
def kernel(h, wg, bg, wd, bd):
    hT = jnp.swapaxes(h, 0, 1)
    wgd = jnp.stack([wg, wd], axis=1)
    bgd = jnp.stack([jnp.asarray(bg, jnp.float32),
                     jnp.asarray(bd, jnp.float32)]).reshape(1, 2)

    fidx, sidx, fused, sused = pl.pallas_call(
        _scan_body,
        out_shape=[
            jax.ShapeDtypeStruct((B, FAST), jnp.int32),
            jax.ShapeDtypeStruct((B, SLOW), jnp.int32),
            jax.ShapeDtypeStruct((B, FAST), jnp.float32),
            jax.ShapeDtypeStruct((B, SLOW), jnp.float32),
        ],
        scratch_shapes=[pltpu.VMEM((T * B, 2), jnp.float32)],
    )(hT, wgd, bgd)

    fast_mem, slow_mem = pl.pallas_call(
        _gather_body,
        grid=(B,),
        in_specs=[
            pl.BlockSpec((1, T, D), lambda b: (b, 0, 0)),
            pl.BlockSpec((B, FAST), lambda b: (0, 0)),
            pl.BlockSpec((B, SLOW), lambda b: (0, 0)),
            pl.BlockSpec((B, FAST), lambda b: (0, 0)),
            pl.BlockSpec((B, SLOW), lambda b: (0, 0)),
        ],
        out_specs=[
            pl.BlockSpec((1, FAST, D), lambda b: (b, 0, 0)),
            pl.BlockSpec((1, SLOW, D), lambda b: (b, 0, 0)),
        ],
        out_shape=[
            jax.ShapeDtypeStruct((B, FAST, D), jnp.float32),
            jax.ShapeDtypeStruct((B, SLOW, D), jnp.float32),
        ],
    )(h, fidx, sidx, fused, sused)

    return fast_mem, slow_mem, fused, sused


# speedup vs baseline: 36.0911x; 1.2746x over previous
import functools

import jax
import jax.numpy as jnp
from jax import lax
from jax.experimental import pallas as pl
from jax.experimental.pallas import tpu as pltpu

D = 512
FAST = 16
SLOW = 64
B = 32
T = 128
STEPS = T - 3
_HI = lax.Precision.HIGHEST


def _scan_body(h_ref, wgd_ref, bgd_ref, fidx_ref, sidx_ref, fused_ref,
               sused_ref, sc_ref):
    bgd = bgd_ref[...]
    wgd_bf = wgd_ref[...].astype(jnp.bfloat16)
    v = h_ref[...].reshape(B * T, D).astype(jnp.bfloat16)
    s = jnp.dot(v, wgd_bf, preferred_element_type=jnp.float32) + bgd
    ws = jax.nn.sigmoid(s[:, 0:1])
    comb = jnp.concatenate([ws, s[:, 1:2]], 1)
    sc_ref[...] = jnp.swapaxes(comb.reshape(B, T, 2), 0, 1).reshape(T * B, 2)

    iota16 = lax.broadcasted_iota(jnp.int32, (B, FAST), 1)
    iota64 = lax.broadcasted_iota(jnp.int32, (B, SLOW), 1)

    def step(t, carry):
        fast_score, fast_tok, slow_tok, nfast, ndem = carry
        s = sc_ref[pl.ds(t * B, B), :]
        w = s[:, 0:1] >= 0.4
        d = s[:, 1:2]
        full = nfast >= FAST
        m = jnp.min(fast_score, axis=1, keepdims=True)
        jmin = jnp.min(jnp.where(fast_score == m, iota16, FAST), axis=1,
                       keepdims=True)
        slot = jnp.where(full, jmin, nfast)
        onehot_f = (iota16 == slot) & w
        victim = jnp.sum(jnp.where(iota16 == jmin, fast_tok, 0), axis=1,
                         keepdims=True)
        demote = w & full
        ring = jnp.bitwise_and(ndem, SLOW - 1)
        onehot_s = (iota64 == ring) & demote
        slow_tok = jnp.where(onehot_s, victim, slow_tok)
        fast_score = jnp.where(onehot_f, d, fast_score)
        fast_tok = jnp.where(onehot_f, t, fast_tok)
        nfast = nfast + (w & ~full).astype(jnp.int32)
        ndem = ndem + demote.astype(jnp.int32)
        return fast_score, fast_tok, slow_tok, nfast, ndem

    init = (jnp.zeros((B, FAST), jnp.float32), jnp.zeros((B, FAST), jnp.int32),
            jnp.zeros((B, SLOW), jnp.int32), jnp.zeros((B, 1), jnp.int32),
            jnp.zeros((B, 1), jnp.int32))
    _, fast_tok, slow_tok, nfast, ndem = lax.fori_loop(0, STEPS, step, init)

    fused = iota16 < nfast
    sused = iota64 < jnp.minimum(ndem, SLOW)
    fidx_ref[...] = jnp.where(fused, fast_tok, 0)
    sidx_ref[...] = jnp.where(sused, slow_tok, 0)
    fused_ref[...] = fused.astype(jnp.float32)
    sused_ref[...] = sused.astype(jnp.float32)


def _gather_body(h_ref, fidx_ref, sidx_ref, fused_ref, sused_ref,
                 fout_ref, sout_ref):
    b = pl.program_id(0)
    hb = h_ref[0]
    iota_tf = lax.broadcasted_iota(jnp.int32, (T, FAST), 0)
    iota_ts = lax.broadcasted_iota(jnp.int32, (T, SLOW), 0)
    ftok = fidx_ref[pl.ds(b, 1), :]
    stok = sidx_ref[pl.ds(b, 1), :]
    fmask = fused_ref[pl.ds(b, 1), :]
    smask = sused_ref[pl.ds(b, 1), :]
    pf = jnp.where(iota_tf == ftok, 1.0, 0.0) * fmask
    ps = jnp.where(iota_ts == stok, 1.0, 0.0) * smask
    dn = (((0,), (0,)), ((), ()))
    fout_ref[0] = lax.dot_general(pf, hb, dn, precision=_HI)
    sout_ref[0] = lax.dot_general(ps, hb, dn, precision=_HI)


@jax.jit
def kernel(h, wg, bg, wd, bd):
    wgd = jnp.stack([wg, wd], axis=1)
    bgd = jnp.stack([jnp.asarray(bg, jnp.float32),
                     jnp.asarray(bd, jnp.float32)]).reshape(1, 2)

    fidx, sidx, fused, sused = pl.pallas_call(
        _scan_body,
        out_shape=[
            jax.ShapeDtypeStruct((B, FAST), jnp.int32),
            jax.ShapeDtypeStruct((B, SLOW), jnp.int32),
            jax.ShapeDtypeStruct((B, FAST), jnp.float32),
            jax.ShapeDtypeStruct((B, SLOW), jnp.float32),
        ],
        scratch_shapes=[pltpu.VMEM((T * B, 2), jnp.float32)],
    )(h, wgd, bgd)

    fast_mem, slow_mem = pl.pallas_call(
        _gather_body,
        grid=(B,),
        in_specs=[
            pl.BlockSpec((1, T, D), lambda b: (b, 0, 0)),
            pl.BlockSpec((B, FAST), lambda b: (0, 0)),
            pl.BlockSpec((B, SLOW), lambda b: (0, 0)),
            pl.BlockSpec((B, FAST), lambda b: (0, 0)),
            pl.BlockSpec((B, SLOW), lambda b: (0, 0)),
        ],
        out_specs=[
            pl.BlockSpec((1, FAST, D), lambda b: (b, 0, 0)),
            pl.BlockSpec((1, SLOW, D), lambda b: (b, 0, 0)),
        ],
        out_shape=[
            jax.ShapeDtypeStruct((B, FAST, D), jnp.float32),
            jax.ShapeDtypeStruct((B, SLOW, D), jnp.float32),
        ],
    )(h, fidx, sidx, fused, sused)

    return fast_mem, slow_mem, fused, sused
